# baseline (speedup 1.0000x reference)
import functools

import jax
import jax.numpy as jnp
from jax import lax
from jax.experimental import pallas as pl
from jax.experimental.pallas import tpu as pltpu
from jax.experimental.pallas import tpu_sc as plsc

B, K, D = 64, 512, 256
BK = B * K


def _chunkseq(a):
    return ((a[:, 0:128] + a[:, 128:256]) + a[:, 256:384]) + a[:, 384:512]


def _degrees_row(parts):
    s = jnp.concatenate([_chunkseq(p) for p in parts], axis=0)
    st = jnp.swapaxes(s, 0, 1)
    acc = st[0:8, :]
    for t in range(1, 16):
        acc = acc + st[8 * t:8 * t + 8, :]
    h4 = acc[0:4, :] + acc[4:8, :]
    h2 = h4[0:2, :] + h4[2:4, :]
    return h2[0:1, :] + h2[1:2, :]


_OGRP = 8


def _order_tc_body(adj_ref, out_ref):
    b = pl.program_id(0)
    adj = adj_ref[0]
    deg_row = _degrees_row(
        [adj[p * 128:(p + 1) * 128, :] for p in range(4)])
    ii = lax.broadcasted_iota(jnp.int32, (K, K), 0)
    jj = lax.broadcasted_iota(jnp.int32, (K, K), 1)
    deg_col = jnp.swapaxes(deg_row, 0, 1)
    h = (deg_col > deg_row) | ((deg_col == deg_row) & (ii < jj))
    rank_row = jnp.sum(h.astype(jnp.int32), axis=0, keepdims=True)
    rank_col = jnp.swapaxes(rank_row, 0, 1)
    order_row = jnp.sum(jnp.where(rank_col == jj, ii, 0), axis=0,
                        keepdims=True)
    val = jnp.concatenate([order_row, rank_row, order_row + b * K],
                          axis=0)
    for i in range(_OGRP):
        @pl.when(b % _OGRP == i)
        def _(i=i):
            out_ref[0, i] = val


_order_tc = pl.pallas_call(
    _order_tc_body,
    grid=(B,),
    in_specs=[pl.BlockSpec((1, K, K), lambda b: (b, 0, 0))],
    out_specs=pl.BlockSpec((1, _OGRP, 3, K), lambda b: (b // _OGRP, 0, 0, 0)),
    out_shape=jax.ShapeDtypeStruct((B // _OGRP, _OGRP, 3, K), jnp.int32),
)


_NC, _NS = 2, 16
_NW = _NC * _NS
_ROWS_PER_W = BK // _NW
_CHUNK = 128
_NCHUNK = _ROWS_PER_W // _CHUNK


@functools.cache
def _gather_sc():
    mesh = plsc.VectorSubcoreMesh(core_axis_name="c", subcore_axis_name="s")

    @functools.partial(
        pl.kernel,
        out_type=jax.ShapeDtypeStruct((BK, D), jnp.float32),
        mesh=mesh,
        scratch_types=[
            pltpu.VMEM((_NCHUNK, _CHUNK), jnp.int32),
            pltpu.VMEM((_CHUNK, D), jnp.float32),
            pltpu.VMEM((_CHUNK, D), jnp.float32),
            pltpu.SemaphoreType.DMA,
            pltpu.SemaphoreType.DMA,
            pltpu.SemaphoreType.DMA,
            pltpu.SemaphoreType.DMA,
        ],
    )
    def gather(table_hbm, idx_hbm, out_hbm, idx_v, buf0, buf1,
               sg0, sg1, sw0, sw1):
        wid = lax.axis_index("s") * _NC + lax.axis_index("c")
        base = wid * _ROWS_PER_W
        pltpu.sync_copy(idx_hbm.at[wid], idx_v)
        bufs = (buf0, buf1)
        gsems = (sg0, sg1)
        wsems = (sw0, sw1)
        ghs = [None] * _NCHUNK
        whs = [None] * _NCHUNK
        for c in range(_NCHUNK):
            if c >= 2:
                whs[c - 2].wait()
            ghs[c] = pltpu.async_copy(
                table_hbm.at[idx_v.at[c]], bufs[c % 2], gsems[c % 2])
            if c >= 1:
                ghs[c - 1].wait()
                whs[c - 1] = pltpu.async_copy(
                    bufs[(c - 1) % 2],
                    out_hbm.at[pl.ds(base + (c - 1) * _CHUNK, _CHUNK)],
                    wsems[(c - 1) % 2])
        ghs[_NCHUNK - 1].wait()
        whs[_NCHUNK - 1] = pltpu.async_copy(
            bufs[(_NCHUNK - 1) % 2],
            out_hbm.at[pl.ds(base + (_NCHUNK - 1) * _CHUNK, _CHUNK)],
            wsems[(_NCHUNK - 1) % 2])
        whs[_NCHUNK - 2].wait()
        whs[_NCHUNK - 1].wait()

    return gather


def kernel(slots, adj):
    packed = _order_tc(adj).reshape(B, 3, K)
    order = packed[:, 0]
    reverse_order = packed[:, 1]
    gidx = packed[:, 2].reshape(_NW, _NCHUNK, _CHUNK)
    ordered = _gather_sc()(slots.reshape(BK, D), gidx)
    return ordered.reshape(B, K, D), order, reverse_order


# speedup vs baseline: 1.0838x; 1.0838x over previous
import functools

import jax
import jax.numpy as jnp
from jax import lax
from jax.experimental import pallas as pl
from jax.experimental.pallas import tpu as pltpu
from jax.experimental.pallas import tpu_sc as plsc

B, K, D = 64, 512, 256
BK = B * K


def _chunkseq(a):
    return ((a[:, 0:128] + a[:, 128:256]) + a[:, 256:384]) + a[:, 384:512]


def _degrees_row(parts):
    s = jnp.concatenate([_chunkseq(p) for p in parts], axis=0)
    st = jnp.swapaxes(s, 0, 1)
    acc = st[0:8, :]
    for t in range(1, 16):
        acc = acc + st[8 * t:8 * t + 8, :]
    h4 = acc[0:4, :] + acc[4:8, :]
    h2 = h4[0:2, :] + h4[2:4, :]
    return h2[0:1, :] + h2[1:2, :]


def _order_tc_body(a0_ref, a1_ref, a2_ref, a3_ref, order_ref, rev_ref,
                   gidx_ref):
    b = pl.program_id(0)
    deg_row = _degrees_row(
        [a0_ref[0], a1_ref[0], a2_ref[0], a3_ref[0]])
    ii = lax.broadcasted_iota(jnp.int32, (K, K), 0)
    jj = lax.broadcasted_iota(jnp.int32, (K, K), 1)
    eye = ii == jj
    deg_col = jnp.sum(jnp.where(eye, deg_row, 0.0), axis=1, keepdims=True)
    h = (deg_col > deg_row) | ((deg_col == deg_row) & (ii < jj))
    rank_row = jnp.sum(h.astype(jnp.int32), axis=0, keepdims=True)
    rev_ref[0] = rank_row
    rank_col = jnp.sum(jnp.where(eye, rank_row, 0), axis=1, keepdims=True)
    order_row = jnp.sum(jnp.where(rank_col == jj, ii, 0), axis=0,
                        keepdims=True)
    order_ref[0] = order_row
    gidx_ref[0] = order_row + b * K


_order_tc = pl.pallas_call(
    _order_tc_body,
    grid=(B,),
    in_specs=[
        pl.BlockSpec((1, K // 4, K), lambda b: (b, 0, 0)),
        pl.BlockSpec((1, K // 4, K), lambda b: (b, 1, 0)),
        pl.BlockSpec((1, K // 4, K), lambda b: (b, 2, 0)),
        pl.BlockSpec((1, K // 4, K), lambda b: (b, 3, 0)),
    ],
    out_specs=[
        pl.BlockSpec((1, 1, K), lambda b: (b, 0, 0)),
        pl.BlockSpec((1, 1, K), lambda b: (b, 0, 0)),
        pl.BlockSpec((1, 1, K), lambda b: (b, 0, 0)),
    ],
    out_shape=[
        jax.ShapeDtypeStruct((B, 1, K), jnp.int32),
        jax.ShapeDtypeStruct((B, 1, K), jnp.int32),
        jax.ShapeDtypeStruct((B, 1, K), jnp.int32),
    ],
)


_NC, _NS = 2, 16
_NW = _NC * _NS
_ROWS_PER_W = BK // _NW
_CHUNK = 128
_NCHUNK = _ROWS_PER_W // _CHUNK


@functools.cache
def _gather_sc():
    mesh = plsc.VectorSubcoreMesh(core_axis_name="c", subcore_axis_name="s")

    @functools.partial(
        pl.kernel,
        out_type=jax.ShapeDtypeStruct((BK, D), jnp.float32),
        mesh=mesh,
        scratch_types=[
            pltpu.VMEM((_NCHUNK, _CHUNK), jnp.int32),
            pltpu.VMEM((_CHUNK, D), jnp.float32),
            pltpu.VMEM((_CHUNK, D), jnp.float32),
            pltpu.SemaphoreType.DMA,
            pltpu.SemaphoreType.DMA,
            pltpu.SemaphoreType.DMA,
            pltpu.SemaphoreType.DMA,
        ],
    )
    def gather(table_hbm, idx_hbm, out_hbm, idx_v, buf0, buf1,
               sg0, sg1, sw0, sw1):
        wid = lax.axis_index("s") * _NC + lax.axis_index("c")
        base = wid * _ROWS_PER_W
        pltpu.sync_copy(idx_hbm.at[wid], idx_v)
        bufs = (buf0, buf1)
        gsems = (sg0, sg1)
        wsems = (sw0, sw1)
        ghs = [None] * _NCHUNK
        whs = [None] * _NCHUNK
        for c in range(_NCHUNK):
            if c >= 2:
                whs[c - 2].wait()
            ghs[c] = pltpu.async_copy(
                table_hbm.at[idx_v.at[c]], bufs[c % 2], gsems[c % 2])
            if c >= 1:
                ghs[c - 1].wait()
                whs[c - 1] = pltpu.async_copy(
                    bufs[(c - 1) % 2],
                    out_hbm.at[pl.ds(base + (c - 1) * _CHUNK, _CHUNK)],
                    wsems[(c - 1) % 2])
        ghs[_NCHUNK - 1].wait()
        whs[_NCHUNK - 1] = pltpu.async_copy(
            bufs[(_NCHUNK - 1) % 2],
            out_hbm.at[pl.ds(base + (_NCHUNK - 1) * _CHUNK, _CHUNK)],
            wsems[(_NCHUNK - 1) % 2])
        whs[_NCHUNK - 2].wait()
        whs[_NCHUNK - 1].wait()

    return gather


def kernel(slots, adj):
    order3, rev3, gidx3 = _order_tc(adj, adj, adj, adj)
    order = order3.reshape(B, K)
    reverse_order = rev3.reshape(B, K)
    gidx = gidx3.reshape(_NW, _NCHUNK, _CHUNK)
    ordered = _gather_sc()(slots.reshape(BK, D), gidx)
    return ordered.reshape(B, K, D), order, reverse_order
